# baseline (speedup 1.0000x reference)
import functools

import jax
import jax.numpy as jnp
from jax.experimental import pallas as pl
from jax.experimental.pallas import tpu as pltpu

_BLOCK = 10000


def _body(nodes_ref, pt_ref, before_ref, after_ref):
    w = nodes_ref[...]
    s = jnp.sum(w, axis=1, keepdims=True)
    before_ref[...] = s.astype(jnp.int32)
    after_ref[...] = s.astype(jnp.int32) + 1
    return
    px = pt_ref[:, 0:1]
    py = pt_ref[:, 1:2]

    lanes = jax.lax.broadcasted_iota(jnp.int32, w.shape, 1)
    even = (lanes & 1) == 0

    diff = w - jnp.where(even, px, py)
    sq = diff * diff
    d = sq + pltpu.roll(sq, 255, 1)

    valid = even & (lanes >= 2) & (lanes <= 252)
    dm = jnp.where(valid, d, jnp.inf)
    mval = jnp.min(dm, axis=1, keepdims=True)
    minlane = jnp.min(jnp.where(dm == mval, lanes, 255), axis=1, keepdims=True)

    t = w - pltpu.roll(w, 254, 1)
    tsq = t * t
    u = tsq + pltpu.roll(tsq, 255, 1)

    sel_next = lanes == minlane
    sel_prev = lanes == (minlane - 2)
    dnext = jnp.sum(jnp.where(sel_next, u, 0.0), axis=1, keepdims=True)
    dprev = jnp.sum(jnp.where(sel_prev, u, 0.0), axis=1, keepdims=True)

    min_idx = minlane >> 1
    before = min_idx - jnp.where(dnext < dprev, 0, 1)
    before_ref[...] = before
    after_ref[...] = before + 1


@jax.jit
def _run(nodes2d, point):
    n = nodes2d.shape[0]
    grid = n // _BLOCK
    out_shape = jax.ShapeDtypeStruct((n, 1), jnp.int32)
    before, after = pl.pallas_call(
        _body,
        grid=(grid,),
        in_specs=[
            pl.BlockSpec((_BLOCK, 256), lambda i: (i, 0)),
            pl.BlockSpec((_BLOCK, 2), lambda i: (i, 0)),
        ],
        out_specs=[
            pl.BlockSpec((_BLOCK, 1), lambda i: (i, 0)),
            pl.BlockSpec((_BLOCK, 1), lambda i: (i, 0)),
        ],
        out_shape=[out_shape, out_shape],
        compiler_params=pltpu.CompilerParams(
            dimension_semantics=("arbitrary",),
        ),
    )(nodes2d, point)
    return before.reshape(n), after.reshape(n)


def _tiny_body(pt_ref, o_ref):
    o_ref[...] = pt_ref[...].astype(jnp.int32)


def kernel(line_nodes, point):
    n = point.shape[0]
    nodes2d = line_nodes.reshape(n, 256)
    s = jnp.sum(nodes2d, axis=1).astype(jnp.int32)
    t = pl.pallas_call(
        _tiny_body,
        grid=(50,),
        in_specs=[pl.BlockSpec((n // 50, 2), lambda i: (i, 0))],
        out_specs=pl.BlockSpec((n // 50, 2), lambda i: (i, 0)),
        out_shape=jax.ShapeDtypeStruct((n, 2), jnp.int32),
    )(point)
    return s + t[:, 0], s + t[:, 1]


# speedup vs baseline: 1.8499x; 1.8499x over previous
import jax
import jax.numpy as jnp
from jax.experimental import pallas as pl
from jax.experimental.pallas import tpu as pltpu

_B = 2000


def _body(w_ref, pt_ref, o_ref):
    w = w_ref[...]
    p = pt_ref[...]
    rows = 2 * _B

    lanes = jax.lax.broadcasted_iota(jnp.int32, w.shape, 1)
    srows = jax.lax.broadcasted_iota(jnp.int32, (rows, 1), 0)
    even_s = (srows & 1) == 0

    df = w - p
    sq = df * df
    d = sq + pltpu.roll(sq, rows - 1, 0)

    valid = (lanes >= 1) & (lanes <= 126)
    dm = jnp.where(valid, d, jnp.inf)
    mval = jnp.min(dm, axis=1, keepdims=True)
    minlane = jnp.min(jnp.where(dm == mval, lanes, 127), axis=1, keepdims=True)

    g = pltpu.roll(w, 127, 1) - w
    gsq = g * g
    u = gsq + pltpu.roll(gsq, rows - 1, 0)

    dnext = jnp.sum(jnp.where(lanes == minlane, u, 0.0), axis=1, keepdims=True)
    dprev = jnp.sum(jnp.where(lanes == (minlane - 1), u, 0.0), axis=1, keepdims=True)

    before = minlane - jnp.where(dnext < dprev, 0, 1)
    o_ref[...] = jnp.where(even_s, before, pltpu.roll(before, 1, 0) + 1)


@jax.jit
def _run(q, p2):
    n2 = q.shape[0]
    grid = n2 // (2 * _B)
    o = pl.pallas_call(
        _body,
        grid=(grid,),
        in_specs=[
            pl.BlockSpec((2 * _B, 128), lambda i: (i, 0)),
            pl.BlockSpec((2 * _B, 1), lambda i: (i, 0)),
        ],
        out_specs=pl.BlockSpec((2 * _B, 1), lambda i: (i, 0)),
        out_shape=jax.ShapeDtypeStruct((n2, 1), jnp.int32),
        compiler_params=pltpu.CompilerParams(
            dimension_semantics=("arbitrary",),
        ),
    )(q, p2)
    return o


def kernel(line_nodes, point):
    n = point.shape[0]
    q = line_nodes.transpose(0, 2, 1).reshape(2 * n, 128)
    p2 = point.reshape(2 * n, 1)
    r = _run(q, p2).reshape(n, 2)
    return r[:, 0], r[:, 1]


# speedup vs baseline: 1.8617x; 1.0064x over previous
import jax
import jax.numpy as jnp
import numpy as np
from jax.experimental import pallas as pl
from jax.experimental.pallas import tpu as pltpu

_B = 2000


def _body(w_ref, pt_ref, pen_ref, o_ref):
    w = w_ref[...]
    p = pt_ref[...]
    rows = 2 * _B

    lanes = jax.lax.broadcasted_iota(jnp.int32, w.shape, 1)
    srows = jax.lax.broadcasted_iota(jnp.int32, (rows, 1), 0)
    even_s = (srows & 1) == 0

    df = w - p
    sq = df * df
    dm = sq + pltpu.roll(sq, rows - 1, 0) + pen_ref[...]

    mval = jnp.min(dm, axis=1, keepdims=True)
    eq = dm == mval
    minlane = jnp.min(jnp.where(eq, lanes, 127), axis=1, keepdims=True)

    g = pltpu.roll(w, 127, 1) - w
    gsq = g * g
    u = gsq + pltpu.roll(gsq, rows - 1, 0)

    first = lanes == minlane
    ddiff = jnp.sum(jnp.where(first, u - pltpu.roll(u, 1, 1), 0.0),
                    axis=1, keepdims=True)

    before = minlane - (ddiff >= 0.0).astype(jnp.int32)
    o_ref[...] = jnp.where(even_s, before, pltpu.roll(before, 1, 0) + 1)


@jax.jit
def _run(q, p2, pen):
    n2 = q.shape[0]
    grid = n2 // (2 * _B)
    o = pl.pallas_call(
        _body,
        grid=(grid,),
        in_specs=[
            pl.BlockSpec((2 * _B, 128), lambda i: (i, 0)),
            pl.BlockSpec((2 * _B, 1), lambda i: (i, 0)),
            pl.BlockSpec((1, 128), lambda i: (0, 0)),
        ],
        out_specs=pl.BlockSpec((2 * _B, 1), lambda i: (i, 0)),
        out_shape=jax.ShapeDtypeStruct((n2, 1), jnp.int32),
        compiler_params=pltpu.CompilerParams(
            dimension_semantics=("arbitrary",),
        ),
    )(q, p2, pen)
    return o


_PEN = np.zeros((1, 128), dtype=np.float32)
_PEN[0, 0] = np.inf
_PEN[0, 127] = np.inf


def kernel(line_nodes, point):
    n = point.shape[0]
    q = line_nodes.transpose(0, 2, 1).reshape(2 * n, 128)
    p2 = point.reshape(2 * n, 1)
    r = _run(q, p2, jnp.asarray(_PEN)).reshape(n, 2)
    return r[:, 0], r[:, 1]
